# Initial kernel scaffold; baseline (speedup 1.0000x reference)
import jax
import jax.numpy as jnp
from jax.experimental import pallas as pl


def kernel(x, edge_index, sex, mutation, age, W1, att_src1, att_dst1, bias1, W2, att_src2, att_dst2, bias2, pool1_W, pool1_b, pool2_W, pool2_b, ln_g, ln_b, sex_emb, mut_emb, age_W, age_b, fc1_W, fc1_b, fc2_W, fc2_b, fc3_W, fc3_b):
    raise NotImplementedError("write your pallas kernel here")


# speedup vs baseline: 65.9289x; 65.9289x over previous
import functools

import jax
import jax.numpy as jnp
from jax import lax
from jax.experimental import pallas as pl
from jax.experimental.pallas import tpu as pltpu
from jax.experimental.pallas import tpu_sc as plsc

N = 10000
E = 320000
HD = 4
C = 16
D = HD * C
DIN = 128
FC1, FC2, OUT = 512, 128, 2

NCORES, NSUB = 2, 16
NW = NCORES * NSUB
EPT = E // NW
CH = 80
NCHUNK = EPT // CH
ROWS_PT = 624
TAIL0 = ROWS_PT * NSUB
TAILR = N - TAIL0

BN = 1000


def _pre_kernel(x_ref, w_ref, as_ref, ad_ref, exp_ref, h_ref, att_ref,
                ih_ref, es_ref, x0_ref):
    xb = x_ref[...]
    h = lax.dot_general(xb, w_ref[...], (((1,), (1,)), ((), ())),
                        preferred_element_type=jnp.float32)
    asrc = jnp.dot(h, as_ref[...], preferred_element_type=jnp.float32)
    adst = jnp.dot(h, ad_ref[...], preferred_element_type=jnp.float32)
    al = asrc + adst
    al = jnp.maximum(al, 0.2 * al)
    es = jnp.exp(al)
    es64 = jnp.dot(es, exp_ref[...], preferred_element_type=jnp.float32)
    h_ref[...] = h
    att_ref[...] = jnp.concatenate([asrc, adst], axis=1)
    ih_ref[...] = 0.5 * es64 * h
    es_ref[...] = jnp.concatenate(
        [es, jnp.zeros((BN, 12), jnp.float32)], axis=1)
    x0_ref[...] = jnp.mean(xb, axis=1, keepdims=True)


def _pre(x, W1, As, Ad, EXPM):
    grid = (N // BN,)
    return pl.pallas_call(
        _pre_kernel,
        grid=grid,
        in_specs=[
            pl.BlockSpec((BN, DIN), lambda i: (i, 0)),
            pl.BlockSpec((D, DIN), lambda i: (0, 0)),
            pl.BlockSpec((D, HD), lambda i: (0, 0)),
            pl.BlockSpec((D, HD), lambda i: (0, 0)),
            pl.BlockSpec((HD, D), lambda i: (0, 0)),
        ],
        out_specs=[
            pl.BlockSpec((BN, D), lambda i: (i, 0)),
            pl.BlockSpec((BN, 8), lambda i: (i, 0)),
            pl.BlockSpec((BN, D), lambda i: (i, 0)),
            pl.BlockSpec((BN, 16), lambda i: (i, 0)),
            pl.BlockSpec((BN, 1), lambda i: (i, 0)),
        ],
        out_shape=[
            jax.ShapeDtypeStruct((N, D), jnp.float32),
            jax.ShapeDtypeStruct((N, 8), jnp.float32),
            jax.ShapeDtypeStruct((N, D), jnp.float32),
            jax.ShapeDtypeStruct((N, 16), jnp.float32),
            jax.ShapeDtypeStruct((N, 1), jnp.float32),
        ],
    )(x, W1, As, Ad, EXPM)


def _sc_edge_body(src_hbm, dst_hbm, h_hbm, att_hbm, ih_hbm,
                  ph_hbm, pd_hbm,
                  src_v, dst_v, rows_v, msg_v, atts_v, attd_v, den_v,
                  acc_sh, sem, sem2, sem3):
    c = lax.axis_index("c")
    sid = lax.axis_index("s")
    wid = c * NSUB + sid
    r0 = sid * ROWS_PT
    pltpu.sync_copy(ih_hbm.at[pl.ds(r0, ROWS_PT)],
                    acc_sh.at[pl.ds(r0, ROWS_PT)])

    @pl.when(sid == NSUB - 1)
    def _():
        pltpu.sync_copy(ih_hbm.at[pl.ds(TAIL0, TAILR)],
                        acc_sh.at[pl.ds(TAIL0, TAILR)])

    zero16 = jnp.zeros((16,), jnp.float32)

    def zloop(i, carry):
        den_v[pl.ds(i * 16, 16)] = zero16
        return carry

    lax.fori_loop(0, (N * HD) // 16, zloop, 0)
    plsc.subcore_barrier()

    base = wid * EPT
    lanes = lax.iota(jnp.int32, 16)

    def chunk(g, carry):
        off = base + g * CH
        pltpu.sync_copy(src_hbm.at[pl.ds(off, CH)], src_v)
        pltpu.sync_copy(dst_hbm.at[pl.ds(off, CH)], dst_v)
        cp1 = pltpu.async_copy(h_hbm.at[src_v], rows_v, sem)
        cp2 = pltpu.async_copy(att_hbm.at[src_v], atts_v, sem2)
        cp3 = pltpu.async_copy(att_hbm.at[dst_v], attd_v, sem3)
        cp1.wait()
        cp2.wait()
        cp3.wait()
        for j in range(CH // 16):
            sl = pl.ds(j * 16, 16)
            dv4 = dst_v[sl] * HD
            lid = jnp.full((16,), j * 16, jnp.int32) + lanes
            evs = []
            for k in range(HD):
                a_s = plsc.load_gather(
                    atts_v, [lid, jnp.full((16,), k, jnp.int32)])
                a_d = plsc.load_gather(
                    attd_v, [lid, jnp.full((16,), HD + k, jnp.int32)])
                al = a_s + a_d
                al = jnp.maximum(al, 0.2 * al)
                ek = jnp.exp(al)
                evs.append(ek)
                plsc.addupdate_scatter(den_v, [dv4 + jnp.int32(k)], ek)
            for i in range(16):
                edge = j * 16 + i
                iv = jnp.full((16,), i, jnp.int32)
                for k in range(HD):
                    bk = evs[k].at[iv].get(mode="promise_in_bounds")
                    msg_v[edge, pl.ds(k * 16, 16)] = \
                        rows_v[edge, pl.ds(k * 16, 16)] * bk
        pltpu.sync_copy(msg_v, acc_sh.at[dst_v], add=True)
        return carry

    lax.fori_loop(0, NCHUNK, chunk, 0)
    plsc.subcore_barrier()
    pltpu.sync_copy(acc_sh.at[pl.ds(r0, ROWS_PT)],
                    ph_hbm.at[c].at[pl.ds(r0, ROWS_PT)])
    pltpu.sync_copy(den_v, pd_hbm.at[wid])

    @pl.when(sid == NSUB - 1)
    def _():
        pltpu.sync_copy(acc_sh.at[pl.ds(TAIL0, TAILR)],
                        ph_hbm.at[c].at[pl.ds(TAIL0, TAILR)])


_SC_EDGE_CACHE = []


def _sc_edge(src, dst, h, att, ih):
    if not _SC_EDGE_CACHE:
        _SC_EDGE_CACHE.append(pl.kernel(
            _sc_edge_body,
            out_type=[jax.ShapeDtypeStruct((NCORES, N, D), jnp.float32),
                      jax.ShapeDtypeStruct((NW, N * HD), jnp.float32)],
            mesh=plsc.VectorSubcoreMesh(
                core_axis_name="c", subcore_axis_name="s",
                num_cores=NCORES, num_subcores=NSUB),
            compiler_params=pltpu.CompilerParams(
                needs_layout_passes=False, use_tc_tiling_on_sc=False),
            scratch_types=[
                pltpu.VMEM((CH,), jnp.int32),
                pltpu.VMEM((CH,), jnp.int32),
                pltpu.VMEM((CH, D), jnp.float32),
                pltpu.VMEM((CH, D), jnp.float32),
                pltpu.VMEM((CH, 8), jnp.float32),
                pltpu.VMEM((CH, 8), jnp.float32),
                pltpu.VMEM((N * HD,), jnp.float32),
                pltpu.VMEM_SHARED((N, D), jnp.float32),
                pltpu.SemaphoreType.DMA,
                pltpu.SemaphoreType.DMA,
                pltpu.SemaphoreType.DMA,
            ],
        ))
    return _SC_EDGE_CACHE[0](src, dst, h, att, ih)


def _mid_kernel(ph_ref, pd_ref, es_ref, b1_ref, p1w_ref, w2_ref, as_ref,
                ad_ref, exp_ref, scal_ref,
                x1_ref, h2_ref, att2_ref, ih2_ref, es2_ref):
    num = ph_ref[0] + ph_ref[1]
    den4 = es_ref[:, :HD] + pd_ref[...]
    den64 = jnp.dot(den4, exp_ref[...],
                    preferred_element_type=jnp.float32)
    v = num / den64 + b1_ref[...]
    h1 = jnp.where(v > 0, v, jnp.exp(v) - 1.0)
    x1_ref[...] = jnp.sum(h1 * p1w_ref[...], axis=1,
                          keepdims=True) + scal_ref[0, 0]
    h2 = lax.dot_general(h1, w2_ref[...], (((1,), (1,)), ((), ())),
                         preferred_element_type=jnp.float32)
    asrc = jnp.dot(h2, as_ref[...], preferred_element_type=jnp.float32)
    adst = jnp.dot(h2, ad_ref[...], preferred_element_type=jnp.float32)
    al = asrc + adst
    al = jnp.maximum(al, 0.2 * al)
    es = jnp.exp(al)
    es64 = jnp.dot(es, exp_ref[...], preferred_element_type=jnp.float32)
    h2_ref[...] = h2
    att2_ref[...] = jnp.concatenate([asrc, adst], axis=1)
    ih2_ref[...] = 0.5 * es64 * h2
    es2_ref[...] = jnp.concatenate(
        [es, jnp.zeros((BN, 12), jnp.float32)], axis=1)


def _mid(ph, pd, es1, bias1, pool1_W, pool1_b, W2, As2, Ad2, EXPM):
    grid = (N // BN,)
    return pl.pallas_call(
        _mid_kernel,
        grid=grid,
        in_specs=[
            pl.BlockSpec((NCORES, BN, D), lambda i: (0, i, 0)),
            pl.BlockSpec((BN, HD), lambda i: (i, 0)),
            pl.BlockSpec((BN, 16), lambda i: (i, 0)),
            pl.BlockSpec((1, D), lambda i: (0, 0)),
            pl.BlockSpec((1, D), lambda i: (0, 0)),
            pl.BlockSpec((D, D), lambda i: (0, 0)),
            pl.BlockSpec((D, HD), lambda i: (0, 0)),
            pl.BlockSpec((D, HD), lambda i: (0, 0)),
            pl.BlockSpec((HD, D), lambda i: (0, 0)),
            pl.BlockSpec((1, 1), lambda i: (0, 0), memory_space=pltpu.SMEM),
        ],
        out_specs=[
            pl.BlockSpec((BN, 1), lambda i: (i, 0)),
            pl.BlockSpec((BN, D), lambda i: (i, 0)),
            pl.BlockSpec((BN, 8), lambda i: (i, 0)),
            pl.BlockSpec((BN, D), lambda i: (i, 0)),
            pl.BlockSpec((BN, 16), lambda i: (i, 0)),
        ],
        out_shape=[
            jax.ShapeDtypeStruct((N, 1), jnp.float32),
            jax.ShapeDtypeStruct((N, D), jnp.float32),
            jax.ShapeDtypeStruct((N, 8), jnp.float32),
            jax.ShapeDtypeStruct((N, D), jnp.float32),
            jax.ShapeDtypeStruct((N, 16), jnp.float32),
        ],
    )(ph, pd, es1, bias1.reshape(1, D), pool1_W,
      W2, As2, Ad2, EXPM, pool1_b.reshape(1, 1))


def _post_kernel(ph_ref, pd_ref, es_ref, b2_ref, p2w_ref, exp_ref, scal_ref,
                 x2_ref):
    num = ph_ref[0] + ph_ref[1]
    den4 = es_ref[:, :HD] + pd_ref[...]
    den64 = jnp.dot(den4, exp_ref[...],
                    preferred_element_type=jnp.float32)
    v = num / den64 + b2_ref[...]
    h2 = jnp.where(v > 0, v, jnp.exp(v) - 1.0)
    x2_ref[...] = jnp.sum(h2 * p2w_ref[...], axis=1,
                          keepdims=True) + scal_ref[0, 0]


def _post(ph, pd, es2, bias2, pool2_W, pool2_b, EXPM):
    grid = (N // BN,)
    return pl.pallas_call(
        _post_kernel,
        grid=grid,
        in_specs=[
            pl.BlockSpec((NCORES, BN, D), lambda i: (0, i, 0)),
            pl.BlockSpec((BN, HD), lambda i: (i, 0)),
            pl.BlockSpec((BN, 16), lambda i: (i, 0)),
            pl.BlockSpec((1, D), lambda i: (0, 0)),
            pl.BlockSpec((1, D), lambda i: (0, 0)),
            pl.BlockSpec((HD, D), lambda i: (0, 0)),
            pl.BlockSpec((1, 1), lambda i: (0, 0), memory_space=pltpu.SMEM),
        ],
        out_specs=pl.BlockSpec((BN, 1), lambda i: (i, 0)),
        out_shape=jax.ShapeDtypeStruct((N, 1), jnp.float32),
    )(ph, pd, es2, bias2.reshape(1, D), pool2_W, EXPM,
      pool2_b.reshape(1, 1))


_RB = 32
_NSTEPS = FC1 // _RB


def _fc_head_kernel(xrows_ref, g_ref, b_ref, w1_ref, b1_ref, w2_ref, b2_ref,
                    w3_ref, b3_ref, out_ref, tot_ref, z1_ref):
    s = pl.program_id(0)

    @pl.when(s == 0)
    def _():
        g = g_ref[...]
        b = b_ref[...]
        for r in range(6):
            v = xrows_ref[r]
            if r >= 3:
                mu = jnp.mean(v)
                var = jnp.mean((v - mu) ** 2)
                v = (v - mu) * jax.lax.rsqrt(var + 1e-5) * g + b
            tot_ref[r] = v

    acc = jnp.zeros((1, _RB), jnp.float32)
    for r in range(6):
        acc = acc + jax.lax.dot_general(
            tot_ref[r], w1_ref[:, r, :], (((1,), (1,)), ((), ())),
            preferred_element_type=jnp.float32)
    z1_ref[s] = acc

    @pl.when(s == _NSTEPS - 1)
    def _():
        acc2 = b2_ref[...]
        for r in range(_NSTEPS):
            zr = jnp.maximum(z1_ref[r] + b1_ref[r], 0.0)
            acc2 = acc2 + jax.lax.dot_general(
                zr, w2_ref[:, r, :], (((1,), (1,)), ((), ())),
                preferred_element_type=jnp.float32)
        z2 = jnp.maximum(acc2, 0.0)
        out_ref[...] = jax.lax.dot_general(
            z2, w3_ref[...], (((1,), (1,)), ((), ())),
            preferred_element_type=jnp.float32) + b3_ref[...]


def _fc_head(xrows, ln_g, ln_b, fc1_W, fc1_b, fc2_W, fc2_b, fc3_W, fc3_b):
    grid = (_NSTEPS,)
    return pl.pallas_call(
        _fc_head_kernel,
        grid=grid,
        in_specs=[
            pl.BlockSpec((6, 1, N), lambda s: (0, 0, 0)),
            pl.BlockSpec((1, N), lambda s: (0, 0)),
            pl.BlockSpec((1, N), lambda s: (0, 0)),
            pl.BlockSpec((_RB, 6, N), lambda s: (s, 0, 0)),
            pl.BlockSpec((_NSTEPS, 1, _RB), lambda s: (0, 0, 0)),
            pl.BlockSpec((FC2, _NSTEPS, _RB), lambda s: (0, 0, 0)),
            pl.BlockSpec((1, FC2), lambda s: (0, 0)),
            pl.BlockSpec((OUT, FC2), lambda s: (0, 0)),
            pl.BlockSpec((1, OUT), lambda s: (0, 0)),
        ],
        out_specs=pl.BlockSpec((1, OUT), lambda s: (0, 0)),
        out_shape=jax.ShapeDtypeStruct((1, OUT), jnp.float32),
        scratch_shapes=[pltpu.VMEM((6, 1, N), jnp.float32),
                        pltpu.VMEM((_NSTEPS, 1, _RB), jnp.float32)],
    )(xrows, ln_g.reshape(1, N), ln_b.reshape(1, N),
      fc1_W.reshape(FC1, 6, N),
      fc1_b.reshape(_NSTEPS, 1, _RB), fc2_W.reshape(FC2, _NSTEPS, _RB),
      fc2_b.reshape(1, FC2), fc3_W, fc3_b.reshape(1, OUT))


def _att_mats(att_src, att_dst):
    eye = jnp.eye(HD, dtype=jnp.float32)
    As = (att_src[0][:, :, None] * eye[:, None, :]).reshape(D, HD)
    Ad = (att_dst[0][:, :, None] * eye[:, None, :]).reshape(D, HD)
    return As, Ad


def kernel(x, edge_index, sex, mutation, age, W1, att_src1, att_dst1, bias1,
           W2, att_src2, att_dst2, bias2, pool1_W, pool1_b, pool2_W, pool2_b,
           ln_g, ln_b, sex_emb, mut_emb, age_W, age_b, fc1_W, fc1_b, fc2_W,
           fc2_b, fc3_W, fc3_b):
    src = edge_index[0]
    dst = edge_index[1]
    EXPM = jnp.kron(jnp.eye(HD, dtype=jnp.float32),
                    jnp.ones((1, C), jnp.float32))
    As1, Ad1 = _att_mats(att_src1, att_dst1)
    As2, Ad2 = _att_mats(att_src2, att_dst2)

    h1t, att1, ih1, es1, x0 = _pre(x, W1, As1, Ad1, EXPM)
    ph1, pd1 = _sc_edge(src, dst, h1t, att1, ih1)
    pd1s = pd1.reshape(NW, N, HD).sum(axis=0)
    x1, h2t, att2, ih2, es2 = _mid(ph1, pd1s, es1, bias1, pool1_W, pool1_b,
                                   W2, As2, Ad2, EXPM)
    ph2, pd2 = _sc_edge(src, dst, h2t, att2, ih2)
    pd2s = pd2.reshape(NW, N, HD).sum(axis=0)
    x2 = _post(ph2, pd2s, es2, bias2, pool2_W, pool2_b, EXPM)

    sf = sex_emb[sex][0]
    mf = mut_emb[mutation][0]
    af = age[0] * age_W[:, 0] + age_b
    xrows = jnp.stack([sf, mf, af, x0[:, 0], x1[:, 0], x2[:, 0]],
                      axis=0).reshape(6, 1, N)
    return _fc_head(xrows, ln_g, ln_b, fc1_W, fc1_b, fc2_W, fc2_b,
                    fc3_W, fc3_b)
